# baseline (speedup 1.0000x reference)
import numpy as np
import jax
import jax.numpy as jnp
from jax.experimental import pallas as pl

_T, _V, _RNN, _EMB, _IN, _OUT, _GH, _GW = 16, 32, 128, 64, 9, 5, 19, 5
_C1 = _RNN // 2
_C2 = _RNN // 4
_H1, _W1 = 8, 3
_IJ = _H1 * _W1
_K1 = 15
_H2 = 4
_F2 = _C2 * _H2

_ii, _jj = np.meshgrid(np.arange(_H1), np.arange(_W1), indexing="ij")
_kh, _kw = np.meshgrid(np.arange(5), np.arange(3), indexing="ij")
_R = (2 * _ii.reshape(-1)[:, None] + _kh.reshape(-1)[None, :]).astype(np.int32)
_C = (_jj.reshape(-1)[:, None] + _kw.reshape(-1)[None, :]).astype(np.int32)


def _body(x_ref, mx_ref, h0_ref, c0_ref, weT_ref, be_ref, w1r_ref, b1_ref,
          w2m_ref, b2e_ref, e2wT_ref, e2b_ref, wihT_ref, whhT_ref, bg_ref,
          woutT_ref, bo_ref, out_ref):
    h = h0_ref[:]
    c = c0_ref[:]
    weT = weT_ref[:]
    w1r = w1r_ref[:]
    w2m = w2m_ref[:]
    e2wT = e2wT_ref[:]
    wihT = wihT_ref[:]
    whhT = whhT_ref[:]
    b1 = b1_ref[:]
    for t in range(_T):
        frame = x_ref[t]
        inp_emb = jax.nn.relu(
            jnp.dot(frame, weT, preferred_element_type=jnp.float32) + be_ref[:])
        P = jnp.dot(h, w1r, preferred_element_type=jnp.float32)
        s = None
        for k in range(_K1):
            mk = mx_ref[t, k]
            pk = P[:, k * _C1:(k + 1) * _C1]
            term = mk[:, :, None] * pk[:, None, :]
            s = term if s is None else s + term
        t1 = jax.nn.relu(s + b1[None, :, :])
        t1f = t1.reshape(_V, _IJ * _C1)
        t2 = jax.nn.relu(
            jnp.dot(t1f, w2m, preferred_element_type=jnp.float32) + b2e_ref[:])
        temb = jax.nn.relu(
            jnp.dot(t2, e2wT, preferred_element_type=jnp.float32) + e2b_ref[:])
        cat = jnp.concatenate([inp_emb, temb], axis=1)
        gates = (jnp.dot(cat, wihT, preferred_element_type=jnp.float32)
                 + jnp.dot(h, whhT, preferred_element_type=jnp.float32)
                 + bg_ref[:])
        i_g = gates[:, 0:_RNN]
        f_g = gates[:, _RNN:2 * _RNN]
        g_g = gates[:, 2 * _RNN:3 * _RNN]
        o_g = gates[:, 3 * _RNN:4 * _RNN]
        c = jax.nn.sigmoid(f_g) * c + jax.nn.sigmoid(i_g) * jnp.tanh(g_g)
        h = jax.nn.sigmoid(o_g) * jnp.tanh(c)
        out_ref[t] = (jnp.dot(h, woutT_ref[:], preferred_element_type=jnp.float32)
                      + bo_ref[:])


def kernel(x_seq, grids, hidden_states, cell_states, W_embed, b_embed,
           conv1_w, conv1_b, conv2_w, conv2_b, embed2_w, embed2_b,
           W_ih, W_hh, b_ih, b_hh, W_out, b_out):
    f32 = jnp.float32
    mask = (grids != -1.0).astype(f32)
    mx = mask[:, :, _R, _C].transpose(0, 3, 1, 2)
    w1r = conv1_w.transpose(1, 2, 3, 0).reshape(_RNN, _K1 * _C1)
    b1 = jnp.broadcast_to(conv1_b[None, :], (_IJ, _C1))
    w2t = conv2_w.transpose(2, 3, 1, 0)
    w5 = jnp.zeros((_H1, _W1, _C1, _C2, _H2), dtype=f32)
    for i2 in range(_H2):
        w5 = w5.at[i2:i2 + 5, :, :, :, i2].set(w2t)
    w2m = w5.reshape(_IJ * _C1, _F2)
    b2e = jnp.repeat(conv2_b, _H2)[None, :]
    out = pl.pallas_call(
        _body,
        out_shape=jax.ShapeDtypeStruct((_T, _V, _OUT), f32),
    )(
        x_seq, mx, hidden_states, cell_states,
        W_embed.T, b_embed[None, :], w1r, b1,
        w2m, b2e, embed2_w.T, embed2_b[None, :],
        W_ih.T, W_hh.T, (b_ih + b_hh)[None, :],
        W_out.T, b_out[None, :],
    )
    return out


# speedup vs baseline: 1.7659x; 1.7659x over previous
import numpy as np
import jax
import jax.numpy as jnp
from jax.experimental import pallas as pl
from jax.experimental.pallas import tpu as pltpu

_T, _V, _RNN, _EMB, _IN, _OUT, _GH, _GW = 16, 32, 128, 64, 9, 5, 19, 5
_C1 = _RNN // 2
_C2 = _RNN // 4
_H1, _W1 = 8, 3
_IJ = _H1 * _W1
_K1 = 15
_H2 = 4
_F2 = _C2 * _H2

_ii, _jj = np.meshgrid(np.arange(_H1), np.arange(_W1), indexing="ij")
_kh, _kw = np.meshgrid(np.arange(5), np.arange(3), indexing="ij")
_R = (2 * _ii.reshape(-1)[:, None] + _kh.reshape(-1)[None, :]).astype(np.int32)
_C = (_jj.reshape(-1)[:, None] + _kw.reshape(-1)[None, :]).astype(np.int32)

_BATCH_DIMS = (((2,), (1,)), ((0,), (0,)))


def _body(x_ref, mx_ref, h0_ref, c0_ref, weT_ref, be_ref, wbig_ref, b1_ref,
          w2m_ref, b2e_ref, e2wT_ref, e2b_ref, wihA_ref, wihB_ref, bg_ref,
          woutT_ref, bo_ref, out_ref, hbuf_ref):
    f32 = jnp.float32
    h = h0_ref[:]
    c = c0_ref[:]
    wbig = wbig_ref[:]
    w2m = w2m_ref[:]
    e2wT = e2wT_ref[:]
    wihB = wihB_ref[:]
    b1 = b1_ref[:]
    bg = bg_ref[:]
    xall = x_ref[:].reshape(_T * _V, _IN)
    inp_embs = jax.nn.relu(
        jnp.dot(xall, weT_ref[:], preferred_element_type=f32) + be_ref[:])
    gx_all = jnp.dot(inp_embs, wihA_ref[:], preferred_element_type=f32)
    for t in range(_T):
        hp = jnp.dot(h, wbig, preferred_element_type=f32)
        p3 = hp[:, :_K1 * _C1].reshape(_V, _K1, _C1)
        gh = hp[:, _K1 * _C1:]
        mx3 = mx_ref[t]
        t1 = jax.nn.relu(
            jax.lax.dot_general(mx3, p3, _BATCH_DIMS,
                                preferred_element_type=f32) + b1[None, :, :])
        t1f = t1.reshape(_V, _IJ * _C1)
        t2 = jax.nn.relu(
            jnp.dot(t1f, w2m, preferred_element_type=f32) + b2e_ref[:])
        temb = jax.nn.relu(
            jnp.dot(t2, e2wT, preferred_element_type=f32) + e2b_ref[:])
        gates = (gx_all[t * _V:(t + 1) * _V, :]
                 + jnp.dot(temb, wihB, preferred_element_type=f32)
                 + gh + bg)
        i_g = gates[:, 0:_RNN]
        f_g = gates[:, _RNN:2 * _RNN]
        g_g = gates[:, 2 * _RNN:3 * _RNN]
        o_g = gates[:, 3 * _RNN:4 * _RNN]
        c = jax.nn.sigmoid(f_g) * c + jax.nn.sigmoid(i_g) * jnp.tanh(g_g)
        h = jax.nn.sigmoid(o_g) * jnp.tanh(c)
        hbuf_ref[t * _V:(t + 1) * _V, :] = h
    out_all = (jnp.dot(hbuf_ref[:], woutT_ref[:], preferred_element_type=f32)
               + bo_ref[:])
    out_ref[:] = out_all.reshape(_T, _V, _OUT)


def kernel(x_seq, grids, hidden_states, cell_states, W_embed, b_embed,
           conv1_w, conv1_b, conv2_w, conv2_b, embed2_w, embed2_b,
           W_ih, W_hh, b_ih, b_hh, W_out, b_out):
    f32 = jnp.float32
    mask = (grids != -1.0).astype(f32)
    mx = mask[:, :, _R, _C]
    w1r = conv1_w.transpose(1, 2, 3, 0).reshape(_RNN, _K1 * _C1)
    wbig = jnp.concatenate([w1r, W_hh.T], axis=1)
    b1 = jnp.broadcast_to(conv1_b[None, :], (_IJ, _C1))
    w2t = conv2_w.transpose(2, 3, 1, 0)
    w5 = jnp.zeros((_H1, _W1, _C1, _C2, _H2), dtype=f32)
    for i2 in range(_H2):
        w5 = w5.at[i2:i2 + 5, :, :, :, i2].set(w2t)
    w2m = w5.reshape(_IJ * _C1, _F2)
    b2e = jnp.repeat(conv2_b, _H2)[None, :]
    out = pl.pallas_call(
        _body,
        out_shape=jax.ShapeDtypeStruct((_T, _V, _OUT), f32),
        scratch_shapes=[pltpu.VMEM((_T * _V, _RNN), f32)],
    )(
        x_seq, mx, hidden_states, cell_states,
        W_embed.T, b_embed[None, :], wbig, b1,
        w2m, b2e, embed2_w.T, embed2_b[None, :],
        W_ih.T[:_EMB, :], W_ih.T[_EMB:, :],
        (b_ih + b_hh)[None, :],
        W_out.T, b_out[None, :],
    )
    return out


# speedup vs baseline: 1.9111x; 1.0822x over previous
import numpy as np
import jax
import jax.numpy as jnp
from jax.experimental import pallas as pl
from jax.experimental.pallas import tpu as pltpu

_T, _V, _RNN, _EMB, _IN, _OUT, _GH, _GW = 16, 32, 128, 64, 9, 5, 19, 5
_C1 = _RNN // 2
_C2 = _RNN // 4
_H1, _W1 = 8, 3
_IJ = _H1 * _W1
_K1 = 15
_H2 = 4
_F2 = _C2 * _H2

_ii, _jj = np.meshgrid(np.arange(_H1), np.arange(_W1), indexing="ij")
_kh, _kw = np.meshgrid(np.arange(5), np.arange(3), indexing="ij")
_R = (2 * _ii.reshape(-1)[:, None] + _kh.reshape(-1)[None, :]).astype(np.int32)
_C = (_jj.reshape(-1)[:, None] + _kw.reshape(-1)[None, :]).astype(np.int32)

_BATCH_DIMS = (((2,), (1,)), ((0,), (0,)))


def _body(x_ref, mx_ref, h0_ref, c0_ref, weT_ref, be_ref, wbig_ref, b1_ref,
          w2m_ref, b2e_ref, e2wT_ref, e2b_ref, wihA_ref, wihB_ref, bg_ref,
          woutT_ref, bo_ref, out_ref, hbuf_ref):
    f32 = jnp.float32
    bf16 = jnp.bfloat16
    h = h0_ref[:].astype(bf16)
    c = c0_ref[:]
    wbig = wbig_ref[:]
    w2m = w2m_ref[:]
    e2wT = e2wT_ref[:]
    wihB = wihB_ref[:]
    b1 = b1_ref[:]
    bg = bg_ref[:]
    xall = x_ref[:].reshape(_T * _V, _IN)
    inp_embs = jax.nn.relu(
        jnp.dot(xall, weT_ref[:], preferred_element_type=f32) + be_ref[:])
    gx_all = jnp.dot(inp_embs.astype(bf16), wihA_ref[:],
                     preferred_element_type=f32)
    for t in range(_T):
        hp = jnp.dot(h, wbig, preferred_element_type=f32)
        p3 = hp[:, :_K1 * _C1].astype(bf16).reshape(_V, _K1, _C1)
        gh = hp[:, _K1 * _C1:]
        mx3 = mx_ref[t]
        t1 = jax.nn.relu(
            jax.lax.dot_general(mx3, p3, _BATCH_DIMS,
                                preferred_element_type=f32) + b1[None, :, :])
        t1f = t1.astype(bf16).reshape(_V, _IJ * _C1)
        t2 = jax.nn.relu(
            jnp.dot(t1f, w2m, preferred_element_type=f32) + b2e_ref[:])
        temb = jax.nn.relu(
            jnp.dot(t2.astype(bf16), e2wT, preferred_element_type=f32)
            + e2b_ref[:])
        gates = (gx_all[t * _V:(t + 1) * _V, :]
                 + jnp.dot(temb.astype(bf16), wihB, preferred_element_type=f32)
                 + gh + bg)
        i_g = gates[:, 0:_RNN]
        f_g = gates[:, _RNN:2 * _RNN]
        g_g = gates[:, 2 * _RNN:3 * _RNN]
        o_g = gates[:, 3 * _RNN:4 * _RNN]
        c = jax.nn.sigmoid(f_g) * c + jax.nn.sigmoid(i_g) * jnp.tanh(g_g)
        h_new = jax.nn.sigmoid(o_g) * jnp.tanh(c)
        hbuf_ref[t * _V:(t + 1) * _V, :] = h_new
        h = h_new.astype(bf16)
    out_all = (jnp.dot(hbuf_ref[:], woutT_ref[:], preferred_element_type=f32)
               + bo_ref[:])
    out_ref[:] = out_all.reshape(_T, _V, _OUT)


def kernel(x_seq, grids, hidden_states, cell_states, W_embed, b_embed,
           conv1_w, conv1_b, conv2_w, conv2_b, embed2_w, embed2_b,
           W_ih, W_hh, b_ih, b_hh, W_out, b_out):
    f32 = jnp.float32
    mask = (grids != -1.0).astype(f32)
    mx = mask[:, :, _R, _C]
    w1r = conv1_w.transpose(1, 2, 3, 0).reshape(_RNN, _K1 * _C1)
    wbig = jnp.concatenate([w1r, W_hh.T], axis=1)
    b1 = jnp.broadcast_to(conv1_b[None, :], (_IJ, _C1))
    w2t = conv2_w.transpose(2, 3, 1, 0)
    w5 = jnp.zeros((_H1, _W1, _C1, _C2, _H2), dtype=f32)
    for i2 in range(_H2):
        w5 = w5.at[i2:i2 + 5, :, :, :, i2].set(w2t)
    w2m = w5.reshape(_IJ * _C1, _F2)
    b2e = jnp.repeat(conv2_b, _H2)[None, :]
    out = pl.pallas_call(
        _body,
        out_shape=jax.ShapeDtypeStruct((_T, _V, _OUT), f32),
        scratch_shapes=[pltpu.VMEM((_T * _V, _RNN), f32)],
    )(
        x_seq, mx.astype(jnp.bfloat16), hidden_states, cell_states,
        W_embed.T, b_embed[None, :], wbig.astype(jnp.bfloat16), b1,
        w2m.astype(jnp.bfloat16), b2e, embed2_w.T.astype(jnp.bfloat16),
        embed2_b[None, :],
        W_ih.T[:_EMB, :].astype(jnp.bfloat16),
        W_ih.T[_EMB:, :].astype(jnp.bfloat16),
        (b_ih + b_hh)[None, :],
        W_out.T, b_out[None, :],
    )
    return out


# speedup vs baseline: 2.0812x; 1.0890x over previous
import numpy as np
import jax
import jax.numpy as jnp
from jax.experimental import pallas as pl
from jax.experimental.pallas import tpu as pltpu

_T, _V, _RNN, _EMB, _IN, _OUT, _GH, _GW = 16, 32, 128, 64, 9, 5, 19, 5
_C1 = _RNN // 2
_C2 = _RNN // 4
_H1, _W1 = 8, 3
_IJ = _H1 * _W1
_K1 = 15
_H2 = 4
_F2 = _C2 * _H2
_P = _GH * _GW

_ii, _jj = np.meshgrid(np.arange(_H1), np.arange(_W1), indexing="ij")
_kh, _kw = np.meshgrid(np.arange(5), np.arange(3), indexing="ij")
_RR = 2 * _ii.reshape(-1)[:, None] + _kh.reshape(-1)[None, :]
_CC = _jj.reshape(-1)[:, None] + _kw.reshape(-1)[None, :]
_S_np = np.zeros((_P, _K1 * _IJ), dtype=np.float32)
for _ij in range(_IJ):
    for _k in range(_K1):
        _S_np[_RR[_ij, _k] * _GW + _CC[_ij, _k], _k * _IJ + _ij] = 1.0

_PERM_np = np.zeros((_F2, _F2), dtype=np.float32)
for _o2 in range(_C2):
    for _i2 in range(_H2):
        _PERM_np[_o2 * _H2 + _i2, _i2 * _C2 + _o2] = 1.0

_BATCH_DIMS = (((2,), (1,)), ((0,), (0,)))


def _body(x_ref, g_ref, h0_ref, c0_ref, we_ref, be_ref, w1_ref, b1_ref,
          w2_ref, b2_ref, e2w_ref, e2b_ref, wih_ref, whh_ref, bih_ref,
          bhh_ref, wout_ref, bo_ref, s_ref, perm_ref, out_ref, hbuf_ref):
    f32 = jnp.float32
    bf16 = jnp.bfloat16
    mflat = (g_ref[:] != -1.0).astype(bf16)
    mxall = jnp.dot(mflat, s_ref[:], preferred_element_type=f32)
    mx3 = mxall.astype(bf16).reshape(_T * _V, _K1, _IJ)
    w1rT = jnp.transpose(w1_ref[:], (0, 2, 1)).reshape(
        _C1 * _K1, _RNN).T.astype(bf16)
    whhT = whh_ref[:].T.astype(bf16)
    wbig = jnp.concatenate([w1rT, whhT], axis=1)
    w2band = w2_ref[:].T.astype(bf16).reshape(_C1, _K1, _C2)
    blocks = []
    for i2 in range(_H2):
        top = i2 * _W1
        bot = _IJ - _K1 - top
        parts = []
        if top:
            parts.append(jnp.zeros((_C1, top, _C2), bf16))
        parts.append(w2band)
        if bot:
            parts.append(jnp.zeros((_C1, bot, _C2), bf16))
        blocks.append(jnp.concatenate(parts, axis=1))
    w2m = jnp.concatenate(blocks, axis=2).reshape(_C1 * _IJ, _F2)
    b2e = jnp.concatenate([b2_ref[:]] * _H2, axis=1)
    e2wPT = jnp.dot(e2w_ref[:].astype(bf16), perm_ref[:],
                    preferred_element_type=f32).T.astype(bf16)
    wihT = wih_ref[:].T
    wihA = wihT[0:_EMB, :].astype(bf16)
    wihB = wihT[_EMB:, :].astype(bf16)
    weT = we_ref[:].T
    woutT = wout_ref[:].T
    bg = bih_ref[:] + bhh_ref[:]
    b1 = b1_ref[:][:, :, None]
    inp_embs = jax.nn.relu(
        jnp.dot(x_ref[:], weT, preferred_element_type=f32) + be_ref[:])
    gx_all = jnp.dot(inp_embs.astype(bf16), wihA, preferred_element_type=f32)
    h = h0_ref[:].astype(bf16)
    c = c0_ref[:]
    for t in range(_T):
        hp = jnp.dot(h, wbig, preferred_element_type=f32)
        p3 = hp[:, :_C1 * _K1].astype(bf16).reshape(_V, _C1, _K1)
        gh = hp[:, _C1 * _K1:]
        t1 = jax.nn.relu(
            jax.lax.dot_general(p3, mx3[t * _V:(t + 1) * _V], _BATCH_DIMS,
                                preferred_element_type=f32) + b1)
        t1f = t1.astype(bf16).reshape(_V, _C1 * _IJ)
        t2 = jax.nn.relu(
            jnp.dot(t1f, w2m, preferred_element_type=f32) + b2e)
        temb = jax.nn.relu(
            jnp.dot(t2.astype(bf16), e2wPT, preferred_element_type=f32)
            + e2b_ref[:])
        gates = (gx_all[t * _V:(t + 1) * _V, :]
                 + jnp.dot(temb.astype(bf16), wihB, preferred_element_type=f32)
                 + gh + bg)
        i_g = gates[:, 0:_RNN]
        f_g = gates[:, _RNN:2 * _RNN]
        g_g = gates[:, 2 * _RNN:3 * _RNN]
        o_g = gates[:, 3 * _RNN:4 * _RNN]
        c = jax.nn.sigmoid(f_g) * c + jax.nn.sigmoid(i_g) * jnp.tanh(g_g)
        h_new = jax.nn.sigmoid(o_g) * jnp.tanh(c)
        hbuf_ref[t * _V:(t + 1) * _V, :] = h_new
        h = h_new.astype(bf16)
    out_all = (jnp.dot(hbuf_ref[:], woutT, preferred_element_type=f32)
               + bo_ref[:])
    out_ref[:] = out_all.reshape(_T, _V, _OUT)


def kernel(x_seq, grids, hidden_states, cell_states, W_embed, b_embed,
           conv1_w, conv1_b, conv2_w, conv2_b, embed2_w, embed2_b,
           W_ih, W_hh, b_ih, b_hh, W_out, b_out):
    f32 = jnp.float32
    out = pl.pallas_call(
        _body,
        out_shape=jax.ShapeDtypeStruct((_T, _V, _OUT), f32),
        scratch_shapes=[pltpu.VMEM((_T * _V, _RNN), f32)],
    )(
        x_seq.reshape(_T * _V, _IN), grids.reshape(_T * _V, _P),
        hidden_states, cell_states,
        W_embed, b_embed[None, :],
        conv1_w.reshape(_C1, _RNN, _K1), conv1_b[None, :],
        conv2_w.reshape(_C2, _C1 * _K1), conv2_b[None, :],
        embed2_w, embed2_b[None, :],
        W_ih, W_hh, b_ih[None, :], b_hh[None, :],
        W_out, b_out[None, :],
        jnp.asarray(_S_np, jnp.bfloat16), jnp.asarray(_PERM_np, jnp.bfloat16),
    )
    return out


# speedup vs baseline: 2.6428x; 1.2698x over previous
import numpy as np
import jax
import jax.numpy as jnp
from jax.experimental import pallas as pl
from jax.experimental.pallas import tpu as pltpu

_T, _V, _RNN, _EMB, _IN, _OUT, _GH, _GW = 16, 32, 128, 64, 9, 5, 19, 5
_C1 = _RNN // 2
_C2 = _RNN // 4
_H1, _W1 = 8, 3
_IJ = _H1 * _W1
_K1 = 15
_H2 = 4
_F2 = _C2 * _H2
_P = _GH * _GW

_ii, _jj = np.meshgrid(np.arange(_H1), np.arange(_W1), indexing="ij")
_kh, _kw = np.meshgrid(np.arange(5), np.arange(3), indexing="ij")
_RR = 2 * _ii.reshape(-1)[:, None] + _kh.reshape(-1)[None, :]
_CC = _jj.reshape(-1)[:, None] + _kw.reshape(-1)[None, :]
_S_np = np.zeros((_P, _IJ * _K1), dtype=np.float32)
for _ij in range(_IJ):
    for _k in range(_K1):
        _S_np[_RR[_ij, _k] * _GW + _CC[_ij, _k], _ij * _K1 + _k] = 1.0

_PERM_np = np.zeros((_F2, _F2), dtype=np.float32)
for _o2 in range(_C2):
    for _i2 in range(_H2):
        _PERM_np[_o2 * _H2 + _i2, _i2 * _C2 + _o2] = 1.0

_BATCH_DIMS = (((2,), (1,)), ((0,), (0,)))


def _body(x_ref, g_ref, h0_ref, c0_ref, we_ref, be_ref, w1_ref, b1_ref,
          w2_ref, b2_ref, e2w_ref, e2b_ref, wih_ref, whh_ref, bih_ref,
          bhh_ref, wout_ref, bo_ref, s_ref, perm_ref, out_ref, hbuf_ref):
    f32 = jnp.float32
    bf16 = jnp.bfloat16
    mflat = (g_ref[:] != -1.0).astype(bf16)
    mxall = jnp.dot(mflat, s_ref[:], preferred_element_type=f32)
    mx3 = mxall.astype(bf16).reshape(_T * _V, _IJ, _K1)
    w1rT = w1_ref[:].T.reshape(_RNN, _K1, _C1).astype(bf16).reshape(
        _RNN, _K1 * _C1)
    whhT = whh_ref[:].T.astype(bf16)
    wbig = jnp.concatenate([w1rT, whhT], axis=1)
    w2band = jnp.transpose(w2_ref[:], (0, 2, 1)).reshape(
        _C2, _K1 * _C1).T.astype(bf16)
    blocks = []
    for i2 in range(_H2):
        top = i2 * _W1 * _C1
        bot = (_H2 - 1) * _W1 * _C1 - top
        parts = []
        if top:
            parts.append(jnp.zeros((top, _C2), bf16))
        parts.append(w2band)
        if bot:
            parts.append(jnp.zeros((bot, _C2), bf16))
        blocks.append(jnp.concatenate(parts, axis=0))
    w2m = jnp.concatenate(blocks, axis=1)
    b2e = jnp.concatenate([b2_ref[:]] * _H2, axis=1)
    e2wPT = jnp.dot(e2w_ref[:].astype(bf16), perm_ref[:],
                    preferred_element_type=f32).T.astype(bf16)
    wihT = wih_ref[:].T
    wihA = wihT[0:_EMB, :].astype(bf16)
    wihB = wihT[_EMB:, :].astype(bf16)
    weT = we_ref[:].T
    woutT = wout_ref[:].T
    bg = bih_ref[:] + bhh_ref[:]
    b1 = b1_ref[:][:, None, :]
    inp_embs = jax.nn.relu(
        jnp.dot(x_ref[:], weT, preferred_element_type=f32) + be_ref[:])
    gx_all = jnp.dot(inp_embs.astype(bf16), wihA, preferred_element_type=f32)
    h = h0_ref[:].astype(bf16)
    c = c0_ref[:]
    for t in range(_T):
        hp = jnp.dot(h, wbig, preferred_element_type=f32)
        p3 = hp[:, :_C1 * _K1].astype(bf16).reshape(_V, _K1, _C1)
        gh = hp[:, _C1 * _K1:]
        t1 = jax.nn.relu(
            jax.lax.dot_general(mx3[t * _V:(t + 1) * _V], p3, _BATCH_DIMS,
                                preferred_element_type=f32) + b1)
        t1f = t1.astype(bf16).reshape(_V, _IJ * _C1)
        t2 = jax.nn.relu(
            jnp.dot(t1f, w2m, preferred_element_type=f32) + b2e)
        temb = jax.nn.relu(
            jnp.dot(t2.astype(bf16), e2wPT, preferred_element_type=f32)
            + e2b_ref[:])
        gates = (gx_all[t * _V:(t + 1) * _V, :]
                 + jnp.dot(temb.astype(bf16), wihB, preferred_element_type=f32)
                 + gh + bg)
        i_g = gates[:, 0:_RNN]
        f_g = gates[:, _RNN:2 * _RNN]
        g_g = gates[:, 2 * _RNN:3 * _RNN]
        o_g = gates[:, 3 * _RNN:4 * _RNN]
        c = jax.nn.sigmoid(f_g) * c + jax.nn.sigmoid(i_g) * jnp.tanh(g_g)
        h_new = jax.nn.sigmoid(o_g) * jnp.tanh(c)
        hbuf_ref[t * _V:(t + 1) * _V, :] = h_new
        h = h_new.astype(bf16)
    out_all = (jnp.dot(hbuf_ref[:], woutT, preferred_element_type=f32)
               + bo_ref[:])
    out_ref[:] = out_all.reshape(_T, _V, _OUT)


def kernel(x_seq, grids, hidden_states, cell_states, W_embed, b_embed,
           conv1_w, conv1_b, conv2_w, conv2_b, embed2_w, embed2_b,
           W_ih, W_hh, b_ih, b_hh, W_out, b_out):
    f32 = jnp.float32
    out = pl.pallas_call(
        _body,
        out_shape=jax.ShapeDtypeStruct((_T, _V, _OUT), f32),
        scratch_shapes=[pltpu.VMEM((_T * _V, _RNN), f32)],
    )(
        x_seq.reshape(_T * _V, _IN), grids.reshape(_T * _V, _P),
        hidden_states, cell_states,
        W_embed, b_embed[None, :],
        conv1_w.reshape(_C1, _RNN * _K1), conv1_b[None, :],
        conv2_w.reshape(_C2, _C1, _K1), conv2_b[None, :],
        embed2_w, embed2_b[None, :],
        W_ih, W_hh, b_ih[None, :], b_hh[None, :],
        W_out, b_out[None, :],
        jnp.asarray(_S_np, jnp.bfloat16), jnp.asarray(_PERM_np, jnp.bfloat16),
    )
    return out
